# baseline (speedup 1.0000x reference)
import functools

import jax
import jax.numpy as jnp
from jax import lax
from jax.experimental import pallas as pl
from jax.experimental.pallas import tpu as pltpu
from jax.experimental.pallas import tpu_sc as plsc

_NUM_CORES = 2
_NUM_SUBCORES = 16
_NW = _NUM_CORES * _NUM_SUBCORES
_RCHUNK = 16
_NSTG = 5
_LANES = 16


def _windows(ncols):
    w = list(range(0, ncols - _LANES + 1, _LANES))
    if ncols % _LANES:
        w.append(ncols - _LANES)
    return w


def _stage_table(pop_hbm, table_sh, stg_v, sem_a, sem_b, sid, nitems):
    slc = -(-nitems // _NUM_SUBCORES)
    slc = -(-slc // (8 * _NSTG)) * (8 * _NSTG)
    stg = slc // _NSTG
    sbase = pl.multiple_of(sid * slc, 8)

    def load(k, b):
        o = jnp.minimum(sbase + k * stg, nitems - stg)
        return pltpu.async_copy(pop_hbm.at[pl.ds(pl.multiple_of(o, 8), stg)],
                                stg_v[b], sem_a[b])

    def store(k, b):
        o = jnp.minimum(sbase + k * stg, nitems - stg)
        return pltpu.async_copy(stg_v[b],
                                table_sh.at[pl.ds(pl.multiple_of(o, 8), stg)],
                                sem_b[b])

    ld = [load(0, 0), None]
    st = [None, None]
    for k in range(_NSTG):
        b = k % 2
        ld[b].wait()
        st[b] = store(k, b)
        if k + 1 < _NSTG:
            if st[1 - b] is not None:
                st[1 - b].wait()
            ld[1 - b] = load(k + 1, 1 - b)
    for cp in st:
        if cp is not None:
            cp.wait()
    plsc.subcore_barrier()


def _gather_body(pop_hbm, batch_hbm, out_hbm, *refs, nrows, ncols, nitems):
    table_sh = refs[0]
    stg_v = refs[1:3]
    idx2d = refs[3:5]
    idx1d = refs[5:7]
    rows1d = refs[7:9]
    sem_i = refs[9:11]
    sem_g = refs[11:13]
    sem_s = refs[13:15]
    rows_per_w = nrows // _NW
    chunks = rows_per_w // _RCHUNK
    n = _RCHUNK * ncols
    sid = lax.axis_index("s")
    wid = sid * _NUM_CORES + lax.axis_index("c")
    rbase = wid * rows_per_w
    last = chunks - 1
    win = _windows(ncols)

    _stage_table(pop_hbm, table_sh, stg_v, sem_i, sem_s, sid, nitems)

    def idx_load(c, b):
        r0 = jnp.minimum(rbase + c * _RCHUNK, rbase + last * _RCHUNK)
        return pltpu.async_copy(batch_hbm.at[pl.ds(r0, _RCHUNK)],
                                idx2d[b], sem_i[b])

    def idx_wait(b):
        pltpu.make_async_copy(batch_hbm.at[pl.ds(rbase, _RCHUNK)],
                              idx2d[b], sem_i[b]).wait()

    def flatten(b):
        for r in range(_RCHUNK):
            for c in win:
                idx1d[b][pl.ds(r * ncols + c, _LANES)] = \
                    idx2d[b][r, pl.ds(c, _LANES)]

    def gather(b):
        return pltpu.async_copy(table_sh.at[idx1d[b]], rows1d[b], sem_g[b])

    def gather_wait(b):
        pltpu.make_async_copy(table_sh.at[idx1d[b]], rows1d[b],
                              sem_g[b]).wait()

    def out_store(c, b):
        fb = (rbase + c * _RCHUNK) * ncols
        return pltpu.async_copy(
            rows1d[b], out_hbm.at[pl.ds(pl.multiple_of(fb, 8), n)],
            sem_s[b])

    def store_wait(b):
        pltpu.make_async_copy(rows1d[b], out_hbm.at[pl.ds(0, n)],
                              sem_s[b]).wait()

    idx_load(0, 0)
    idx_load(1, 1)
    idx_wait(0)
    flatten(0)
    gather(0)
    idx_load(2, 0)
    idx_wait(1)
    flatten(1)
    gather(1)
    idx_load(3, 1)
    gather_wait(0)
    out_store(0, 0)
    gather_wait(1)
    out_store(1, 1)

    def body(j, carry):
        e = 2 * j
        idx_wait(0)
        flatten(0)
        store_wait(0)
        gather(0)
        idx_load(jnp.minimum(e + 2, last), 0)
        idx_wait(1)
        flatten(1)
        store_wait(1)
        gather(1)
        idx_load(jnp.minimum(e + 3, last), 1)
        gather_wait(0)
        out_store(e, 0)
        gather_wait(1)
        out_store(e + 1, 1)
        return carry

    if chunks > 2:
        lax.fori_loop(1, chunks // 2, body, 0)
    idx_wait(0)
    idx_wait(1)
    store_wait(0)
    store_wait(1)


@functools.partial(jax.jit, static_argnames=("nrows", "ncols", "nitems"))
def _gather(pop, batch, nrows, ncols, nitems):
    mesh = plsc.VectorSubcoreMesh(core_axis_name="c", subcore_axis_name="s")
    slc = -(-nitems // _NUM_SUBCORES)
    slc = -(-slc // (8 * _NSTG)) * (8 * _NSTG)
    stg = slc // _NSTG
    f = functools.partial(
        pl.kernel,
        mesh=mesh,
        out_type=jax.ShapeDtypeStruct((nrows * ncols,), jnp.float32),
        scratch_types=(
            [pltpu.VMEM_SHARED((nitems,), jnp.float32)]
            + [pltpu.VMEM((stg,), jnp.float32) for _ in range(2)]
            + [pltpu.VMEM((_RCHUNK, ncols), jnp.int32) for _ in range(2)]
            + [pltpu.VMEM((_RCHUNK * ncols,), jnp.int32) for _ in range(2)]
            + [pltpu.VMEM((_RCHUNK * ncols,), jnp.float32) for _ in range(2)]
            + [pltpu.SemaphoreType.DMA for _ in range(6)]
        ),
    )(functools.partial(_gather_body, nrows=nrows, ncols=ncols,
                        nitems=nitems))
    return f(pop, batch)


def kernel(pop, batch):
    rows, cols = batch.shape
    out = _gather(pop, batch.astype(jnp.int32), rows, cols, pop.shape[0])
    return out.reshape(rows, cols)
